# baseline (device time: 23615 ns/iter reference)
import jax
import jax.numpy as jnp
from jax import lax
from jax.experimental import pallas as pl
from jax.experimental.pallas import tpu as pltpu

N_DEV = 4
B_LOC = 2
SQ = 128
SKV = 128
DH = 64
D_MODEL = 512
D_QKV = 1024
CHUNK = D_QKV // N_DEV
HALF = CHUNK // 2
MESH = pl.DeviceIdType.MESH


def kernel(x, Wq, K_ext, V_ext, Wo):
    K2 = K_ext.reshape(8, SKV, 16 * DH)
    V2 = V_ext.reshape(8, SKV, 16 * DH)

    def body(x_ref, wq_ref, k_ref, v_ref, wo_ref, out_ref,
             qAr, qBr, oAr, oBr, qAd, qBd, oAd, oBd,
             wqA_bf, wqB_bf, woA_bf, woB_bf,
             x_bf, k_bf, v_bf,
             qAr_s, qAr_r, qBr_s, qBr_r, oAr_s, oAr_r, oBr_s, oBr_r,
             d_s, d_r):
        my_pos = lax.axis_index("i")
        left = lax.rem(my_pos + N_DEV - 1, N_DEV)
        right = lax.rem(my_pos + 1, N_DEV)

        barrier = pltpu.get_barrier_semaphore()
        for nbr in (left, right):
            pl.semaphore_signal(barrier, inc=1, device_id=(nbr,),
                                device_id_type=MESH)
        pl.semaphore_wait(barrier, 2)

        wqA_bf[...] = wq_ref[:, 0:HALF].astype(jnp.bfloat16)
        wqB_bf[...] = wq_ref[:, HALF:CHUNK].astype(jnp.bfloat16)
        woA_bf[...] = wo_ref[0:HALF, :].astype(jnp.bfloat16)
        woB_bf[...] = wo_ref[HALF:CHUNK, :].astype(jnp.bfloat16)

        def rc(src, dst, ssem, rsem, tgt):
            r = pltpu.make_async_remote_copy(
                src_ref=src, dst_ref=dst, send_sem=ssem, recv_sem=rsem,
                device_id=(tgt,), device_id_type=MESH)
            r.start()
            return r

        r_qA0 = rc(wqA_bf, qAr.at[0], qAr_s.at[0], qAr_r.at[0], right)
        r_oA0 = rc(woA_bf, oAr.at[0], oAr_s.at[0], oAr_r.at[0], right)
        r_qB0 = rc(wqB_bf, qBr.at[0], qBr_s.at[0], qBr_r.at[0], left)
        r_oB0 = rc(woB_bf, oBr.at[0], oBr_s.at[0], oBr_r.at[0], left)
        d_qA = rc(wqA_bf, qAd, d_s.at[0], d_r.at[0], left)
        d_oA = rc(woA_bf, oAd, d_s.at[1], d_r.at[1], left)
        d_qB = rc(wqB_bf, qBd, d_s.at[2], d_r.at[2], right)
        d_oB = rc(woB_bf, oBd, d_s.at[3], d_r.at[3], right)

        for b in range(B_LOC):
            gb = my_pos * B_LOC + b
            x_bf[b] = x_ref[b].astype(jnp.bfloat16)
            k_bf[b] = k_ref[gb].astype(jnp.bfloat16)
            v_bf[b] = v_ref[gb].astype(jnp.bfloat16)

        qb = lax.broadcasted_iota(jnp.int32, (SQ, SKV), 0) // 64
        kb = lax.broadcasted_iota(jnp.int32, (SQ, SKV), 1) // 64
        mask = (qb == kb) | ((kb % 4) == (qb % 4))

        def compute_unit(o, s, wq_h, wo_h, init=False):
            col = o * CHUNK + s * HALF
            for b in range(B_LOC):
                q2 = jnp.dot(x_bf[b], wq_h,
                             preferred_element_type=jnp.float32)
                q2 = q2.astype(jnp.bfloat16)
                k2 = k_bf[b, :, pl.ds(col, 2 * DH)]
                v2 = v_bf[b, :, pl.ds(col, 2 * DH)]
                ctxs = []
                for hh in range(2):
                    q = q2[:, hh * DH:(hh + 1) * DH]
                    k = k2[:, hh * DH:(hh + 1) * DH]
                    v = v2[:, hh * DH:(hh + 1) * DH]
                    sm = lax.dot_general(
                        q, k, (((1,), (1,)), ((), ())),
                        preferred_element_type=jnp.float32) * 0.125
                    w = jnp.exp(jnp.where(mask, sm, -1e9))
                    rws = 1.0 / jnp.sum(w, axis=-1, keepdims=True)
                    ctxs.append(
                        jnp.dot(w.astype(jnp.bfloat16), v,
                                preferred_element_type=jnp.float32)
                        * rws)
                ctx2 = jnp.concatenate(ctxs, axis=1)
                contrib = jnp.dot(ctx2.astype(jnp.bfloat16), wo_h,
                                  preferred_element_type=jnp.float32)
                if init:
                    out_ref[b] = contrib
                else:
                    out_ref[b] = out_ref[b] + contrib

        compute_unit(my_pos, 0, wqA_bf[...], woA_bf[...], init=True)
        r_qA0.wait_recv()
        r_oA0.wait_recv()
        f_qA = rc(qAr.at[0], qAr.at[1], qAr_s.at[1], qAr_r.at[1], right)
        f_oA = rc(oAr.at[0], oAr.at[1], oAr_s.at[1], oAr_r.at[1], right)
        compute_unit(my_pos, 1, wqB_bf[...], woB_bf[...])
        r_qB0.wait_recv()
        r_oB0.wait_recv()
        f_qB = rc(qBr.at[0], qBr.at[1], qBr_s.at[1], qBr_r.at[1], left)
        f_oB = rc(oBr.at[0], oBr.at[1], oBr_s.at[1], oBr_r.at[1], left)

        compute_unit(left, 0, qAr[0], oAr[0])
        compute_unit(right, 1, qBr[0], oBr[0])

        d_qA.wait_recv()
        d_oA.wait_recv()
        compute_unit(right, 0, qAd[...], oAd[...])
        d_qB.wait_recv()
        d_oB.wait_recv()
        compute_unit(left, 1, qBd[...], oBd[...])

        opp = lax.rem(my_pos + 2, N_DEV)
        f_qA.wait_recv()
        f_oA.wait_recv()
        compute_unit(opp, 0, qAr[1], oAr[1])
        f_qB.wait_recv()
        f_oB.wait_recv()
        compute_unit(opp, 1, qBr[1], oBr[1])

        for r in (r_qA0, r_oA0, r_qB0, r_oB0, d_qA, d_oA, d_qB, d_oB,
                  f_qA, f_oA, f_qB, f_oB):
            r.wait_send()

    return pl.pallas_call(
        body,
        out_shape=jax.ShapeDtypeStruct((B_LOC, SQ, D_MODEL), jnp.float32),
        in_specs=[pl.BlockSpec(memory_space=pltpu.VMEM)] * 5,
        out_specs=pl.BlockSpec(memory_space=pltpu.VMEM),
        scratch_shapes=[
            pltpu.VMEM((2, D_MODEL, HALF), jnp.bfloat16),
            pltpu.VMEM((2, D_MODEL, HALF), jnp.bfloat16),
            pltpu.VMEM((2, HALF, D_MODEL), jnp.bfloat16),
            pltpu.VMEM((2, HALF, D_MODEL), jnp.bfloat16),
            pltpu.VMEM((D_MODEL, HALF), jnp.bfloat16),
            pltpu.VMEM((D_MODEL, HALF), jnp.bfloat16),
            pltpu.VMEM((HALF, D_MODEL), jnp.bfloat16),
            pltpu.VMEM((HALF, D_MODEL), jnp.bfloat16),
            pltpu.VMEM((D_MODEL, HALF), jnp.bfloat16),
            pltpu.VMEM((D_MODEL, HALF), jnp.bfloat16),
            pltpu.VMEM((HALF, D_MODEL), jnp.bfloat16),
            pltpu.VMEM((HALF, D_MODEL), jnp.bfloat16),
            pltpu.VMEM((B_LOC, SQ, D_MODEL), jnp.bfloat16),
            pltpu.VMEM((B_LOC, SKV, 16 * DH), jnp.bfloat16),
            pltpu.VMEM((B_LOC, SKV, 16 * DH), jnp.bfloat16),
            pltpu.SemaphoreType.DMA((2,)),
            pltpu.SemaphoreType.DMA((2,)),
            pltpu.SemaphoreType.DMA((2,)),
            pltpu.SemaphoreType.DMA((2,)),
            pltpu.SemaphoreType.DMA((2,)),
            pltpu.SemaphoreType.DMA((2,)),
            pltpu.SemaphoreType.DMA((2,)),
            pltpu.SemaphoreType.DMA((2,)),
            pltpu.SemaphoreType.DMA((4,)),
            pltpu.SemaphoreType.DMA((4,)),
        ],
        compiler_params=pltpu.CompilerParams(collective_id=0),
    )(x, Wq, K2, V2, Wo)


# device time: 19857 ns/iter; 1.1893x vs baseline; 1.1893x over previous
import jax
import jax.numpy as jnp
from jax import lax
from jax.experimental import pallas as pl
from jax.experimental.pallas import tpu as pltpu

N_DEV = 4
B_LOC = 2
SQ = 128
SKV = 128
DH = 64
D_MODEL = 512
D_QKV = 1024
CHUNK = D_QKV // N_DEV
HALF = CHUNK // 2
MESH = pl.DeviceIdType.MESH


def kernel(x, Wq, K_ext, V_ext, Wo):
    idx = lax.axis_index("i")
    K2 = lax.dynamic_slice_in_dim(K_ext.reshape(8, SKV, 16 * DH),
                                  idx * B_LOC, B_LOC, 0)
    V2 = lax.dynamic_slice_in_dim(V_ext.reshape(8, SKV, 16 * DH),
                                  idx * B_LOC, B_LOC, 0)

    def body(x_ref, wq_ref, k_ref, v_ref, wo_ref, out_ref,
             qAr, qBr, oAr, oBr, qAd, qBd, oAd, oBd,
             wqA_bf, wqB_bf, woA_bf, woB_bf,
             qAr_s, qAr_r, qBr_s, qBr_r, oAr_s, oAr_r, oBr_s, oBr_r,
             d_s, d_r):
        my_pos = lax.axis_index("i")
        left = lax.rem(my_pos + N_DEV - 1, N_DEV)
        right = lax.rem(my_pos + 1, N_DEV)

        barrier = pltpu.get_barrier_semaphore()
        for nbr in (left, right):
            pl.semaphore_signal(barrier, inc=1, device_id=(nbr,),
                                device_id_type=MESH)
        pl.semaphore_wait(barrier, 2)

        wqA_bf[...] = wq_ref[:, 0:HALF].astype(jnp.bfloat16)
        wqB_bf[...] = wq_ref[:, HALF:CHUNK].astype(jnp.bfloat16)
        woA_bf[...] = wo_ref[0:HALF, :].astype(jnp.bfloat16)
        woB_bf[...] = wo_ref[HALF:CHUNK, :].astype(jnp.bfloat16)

        def rc(src, dst, ssem, rsem, tgt):
            r = pltpu.make_async_remote_copy(
                src_ref=src, dst_ref=dst, send_sem=ssem, recv_sem=rsem,
                device_id=(tgt,), device_id_type=MESH)
            r.start()
            return r

        r_qA0 = rc(wqA_bf, qAr.at[0], qAr_s.at[0], qAr_r.at[0], right)
        r_oA0 = rc(woA_bf, oAr.at[0], oAr_s.at[0], oAr_r.at[0], right)
        r_qB0 = rc(wqB_bf, qBr.at[0], qBr_s.at[0], qBr_r.at[0], left)
        r_oB0 = rc(woB_bf, oBr.at[0], oBr_s.at[0], oBr_r.at[0], left)
        d_qA = rc(wqA_bf, qAd, d_s.at[0], d_r.at[0], left)
        d_oA = rc(woA_bf, oAd, d_s.at[1], d_r.at[1], left)
        d_qB = rc(wqB_bf, qBd, d_s.at[2], d_r.at[2], right)
        d_oB = rc(woB_bf, oBd, d_s.at[3], d_r.at[3], right)

        qb = lax.broadcasted_iota(jnp.int32, (SQ, SKV), 0) // 64
        kb = lax.broadcasted_iota(jnp.int32, (SQ, SKV), 1) // 64
        mask = (qb == kb) | ((kb % 4) == (qb % 4))

        def compute_unit(o, s, wq_h, wo_h, init=False):
            col = o * CHUNK + s * HALF
            wq_h = wq_h.astype(jnp.float32)
            wo_h = wo_h.astype(jnp.float32)
            for b in range(B_LOC):
                q2 = jnp.dot(x_ref[b], wq_h,
                             preferred_element_type=jnp.float32)
                k2 = k_ref[b, :, pl.ds(col, 2 * DH)]
                v2 = v_ref[b, :, pl.ds(col, 2 * DH)]
                ctxs = []
                for hh in range(2):
                    q = q2[:, hh * DH:(hh + 1) * DH]
                    k = k2[:, hh * DH:(hh + 1) * DH]
                    v = v2[:, hh * DH:(hh + 1) * DH]
                    sm = lax.dot_general(
                        q, k, (((1,), (1,)), ((), ())),
                        preferred_element_type=jnp.float32) * 0.125
                    w = jnp.exp(jnp.where(mask, sm, -1e9))
                    rws = 1.0 / jnp.sum(w, axis=-1, keepdims=True)
                    ctxs.append(
                        jnp.dot(w, v, preferred_element_type=jnp.float32)
                        * rws)
                ctx2 = jnp.concatenate(ctxs, axis=1)
                contrib = jnp.dot(ctx2, wo_h,
                                  preferred_element_type=jnp.float32)
                if init:
                    out_ref[b] = contrib
                else:
                    out_ref[b] = out_ref[b] + contrib

        compute_unit(my_pos, 0, wqA_bf[...], woA_bf[...], init=True)
        r_qA0.wait_recv()
        r_oA0.wait_recv()
        f_qA = rc(qAr.at[0], qAr.at[1], qAr_s.at[1], qAr_r.at[1], right)
        f_oA = rc(oAr.at[0], oAr.at[1], oAr_s.at[1], oAr_r.at[1], right)
        compute_unit(my_pos, 1, wqB_bf[...], woB_bf[...])
        r_qB0.wait_recv()
        r_oB0.wait_recv()
        f_qB = rc(qBr.at[0], qBr.at[1], qBr_s.at[1], qBr_r.at[1], left)
        f_oB = rc(oBr.at[0], oBr.at[1], oBr_s.at[1], oBr_r.at[1], left)

        compute_unit(left, 0, qAr[0], oAr[0])
        compute_unit(right, 1, qBr[0], oBr[0])

        d_qA.wait_recv()
        d_oA.wait_recv()
        compute_unit(right, 0, qAd[...], oAd[...])
        d_qB.wait_recv()
        d_oB.wait_recv()
        compute_unit(left, 1, qBd[...], oBd[...])

        opp = lax.rem(my_pos + 2, N_DEV)
        f_qA.wait_recv()
        f_oA.wait_recv()
        compute_unit(opp, 0, qAr[1], oAr[1])
        f_qB.wait_recv()
        f_oB.wait_recv()
        compute_unit(opp, 1, qBr[1], oBr[1])

        for r in (r_qA0, r_oA0, r_qB0, r_oB0, d_qA, d_oA, d_qB, d_oB,
                  f_qA, f_oA, f_qB, f_oB):
            r.wait_send()

    return pl.pallas_call(
        body,
        out_shape=jax.ShapeDtypeStruct((B_LOC, SQ, D_MODEL), jnp.float32),
        in_specs=[pl.BlockSpec(memory_space=pltpu.VMEM)] * 5,
        out_specs=pl.BlockSpec(memory_space=pltpu.VMEM),
        scratch_shapes=[
            pltpu.VMEM((2, D_MODEL, HALF), jnp.bfloat16),
            pltpu.VMEM((2, D_MODEL, HALF), jnp.bfloat16),
            pltpu.VMEM((2, HALF, D_MODEL), jnp.bfloat16),
            pltpu.VMEM((2, HALF, D_MODEL), jnp.bfloat16),
            pltpu.VMEM((D_MODEL, HALF), jnp.bfloat16),
            pltpu.VMEM((D_MODEL, HALF), jnp.bfloat16),
            pltpu.VMEM((HALF, D_MODEL), jnp.bfloat16),
            pltpu.VMEM((HALF, D_MODEL), jnp.bfloat16),
            pltpu.VMEM((D_MODEL, HALF), jnp.bfloat16),
            pltpu.VMEM((D_MODEL, HALF), jnp.bfloat16),
            pltpu.VMEM((HALF, D_MODEL), jnp.bfloat16),
            pltpu.VMEM((HALF, D_MODEL), jnp.bfloat16),
            pltpu.SemaphoreType.DMA((2,)),
            pltpu.SemaphoreType.DMA((2,)),
            pltpu.SemaphoreType.DMA((2,)),
            pltpu.SemaphoreType.DMA((2,)),
            pltpu.SemaphoreType.DMA((2,)),
            pltpu.SemaphoreType.DMA((2,)),
            pltpu.SemaphoreType.DMA((2,)),
            pltpu.SemaphoreType.DMA((2,)),
            pltpu.SemaphoreType.DMA((4,)),
            pltpu.SemaphoreType.DMA((4,)),
        ],
        compiler_params=pltpu.CompilerParams(collective_id=0),
    )(x, Wq, K2, V2, Wo)
